# Initial kernel scaffold; baseline (speedup 1.0000x reference)
import jax
import jax.numpy as jnp
from jax.experimental import pallas as pl


def kernel(x, ln1_w, ln2_w, Wqkv, bqkv, Wproj, bproj, gate_w, gate_b, We1, be1, We2, be2, Ws1, bs1, Ws2, bs2):
    raise NotImplementedError("write your pallas kernel here")


# speedup vs baseline: 3.1913x; 3.1913x over previous
import functools

import jax
import jax.numpy as jnp
from jax.experimental import pallas as pl
from jax.experimental.pallas import tpu as pltpu

B, T, C = 1, 2048, 768
H = 12
HD = C // H
E = 64
TOPK = 2
DFF = 2 * C
THETA = 10000.0
ALPHA = 0.01
EPS = 1e-5

QKV_ROWS = 512
ATT_ROWS = 512
MOE_TILE = 128
SH_ROWS = 512


def _rms(x, w):
    return x * jax.lax.rsqrt(jnp.mean(x * x, axis=-1, keepdims=True) + EPS) * w


def _silu(a):
    return a * (1.0 / (1.0 + jnp.exp(-a)))


def _qkv_kernel(x_ref, ln1_ref, wext_ref, bext_ref, cos_ref, sin_ref,
                q_ref, k_ref, v_ref):
    h = _rms(x_ref[...], ln1_ref[...])
    z = jnp.dot(h, wext_ref[...], preferred_element_type=jnp.float32)
    z = z + bext_ref[...]
    q = z[:, :C]
    k = z[:, C:2 * C]
    v = z[:, 2 * C:3 * C]
    qs = z[:, 3 * C:4 * C]
    ks = z[:, 4 * C:5 * C]
    cos = cos_ref[...]
    sin = sin_ref[...]
    q_ref[...] = q * cos + qs * sin
    k_ref[...] = k * cos + ks * sin
    v_ref[...] = v


def _att_kernel(q_ref, k_ref, v_ref, o_ref):
    i = pl.program_id(1)
    q = q_ref[0]
    s = jax.lax.dot_general(q, k_ref[0], (((1,), (1,)), ((), ())),
                            preferred_element_type=jnp.float32)
    s = s * (1.0 / (HD ** 0.5))
    row = jax.lax.broadcasted_iota(jnp.int32, (ATT_ROWS, T), 0) + i * ATT_ROWS
    col = jax.lax.broadcasted_iota(jnp.int32, (ATT_ROWS, T), 1)
    s = jnp.where(row >= col, s, jnp.float32(-1e30))
    m = jnp.max(s, axis=-1, keepdims=True)
    p = jnp.exp(s - m)
    p = p / jnp.sum(p, axis=-1, keepdims=True)
    o_ref[0] = jnp.dot(p, v_ref[0], preferred_element_type=jnp.float32)


def _post_kernel(x_ref, y_ref, wp_ref, bp_ref, ln2_ref, gw_ref, gb_ref,
                 x1_ref, x2_ref, d1_ref, d2_ref, w1_ref, w2_ref,
                 off_ref, cnt_ref, loss_ref):
    x1 = x_ref[...] + jnp.dot(y_ref[...], wp_ref[...],
                              preferred_element_type=jnp.float32) + bp_ref[...]
    x1_ref[...] = x1
    x2 = _rms(x1, ln2_ref[...])
    x2_ref[...] = x2
    logits = jnp.dot(x2, gw_ref[...], preferred_element_type=jnp.float32)
    lmax = jnp.max(logits, axis=-1, keepdims=True)
    ex = jnp.exp(logits - lmax)
    a = ex / jnp.sum(ex, axis=-1, keepdims=True)
    b = a + gb_ref[...]
    lane = jax.lax.broadcasted_iota(jnp.int32, (T, E), 1)
    m1 = jnp.max(b, axis=-1, keepdims=True)
    i1 = jnp.min(jnp.where(b == m1, lane, E), axis=-1, keepdims=True)
    oh1 = (lane == i1).astype(jnp.float32)
    b2 = jnp.where(lane == i1, jnp.float32(-1e30), b)
    m2 = jnp.max(b2, axis=-1, keepdims=True)
    i2 = jnp.min(jnp.where(b2 == m2, lane, E), axis=-1, keepdims=True)
    oh2 = (lane == i2).astype(jnp.float32)
    w1_ref[...] = jnp.sum(oh1 * a, axis=-1, keepdims=True)
    w2_ref[...] = jnp.sum(oh2 * a, axis=-1, keepdims=True)
    ohs = oh1 + oh2
    counts = jnp.sum(ohs, axis=0, keepdims=True)
    rt = jax.lax.broadcasted_iota(jnp.int32, (T, T), 0)
    ct = jax.lax.broadcasted_iota(jnp.int32, (T, T), 1)
    ltri = (rt > ct).astype(jnp.float32)
    cum = jnp.dot(ltri, ohs, preferred_element_type=jnp.float32)
    re = jax.lax.broadcasted_iota(jnp.int32, (E, E), 0)
    ce = jax.lax.broadcasted_iota(jnp.int32, (E, E), 1)
    upe = (re < ce).astype(jnp.float32)
    offs = jnp.dot(counts, upe, preferred_element_type=jnp.float32)
    base = cum + offs
    d1_ref[...] = jnp.sum(oh1 * base, axis=-1, keepdims=True)
    d2_ref[...] = jnp.sum(oh2 * base, axis=-1, keepdims=True) + \
        jnp.sum(oh1 * oh2, axis=-1, keepdims=True)
    off_ref[...] = offs
    cnt_ref[...] = counts
    probs = jnp.mean(a, axis=0, keepdims=True)
    f_i = counts * jnp.float32(E) / (jnp.float32(TOPK * T) + 1e-6)
    loss_ref[...] = jnp.sum(f_i * probs, keepdims=True).reshape(1, 1) * ALPHA


def _shared_kernel(x1_ref, x2_ref, ws1_ref, bs1_ref, ws2_ref, bs2_ref,
                   o_ref):
    h = _silu(jnp.dot(x2_ref[...], ws1_ref[...],
                      preferred_element_type=jnp.float32) + bs1_ref[...])
    o_ref[...] = x1_ref[...] + jnp.dot(h, ws2_ref[...],
                                       preferred_element_type=jnp.float32) \
        + bs2_ref[...]


def _moe_kernel(sc_ref, x2_ref, d1_ref, d2_ref, w1_ref, w2_ref, base_ref,
                we1_ref, be1_ref, we2_ref, be2_ref, out_ref):
    e = pl.program_id(0)
    off = sc_ref[e]
    cnt = sc_ref[E + e]

    @pl.when(e == 0)
    def _():
        out_ref[...] = base_ref[...]

    w1m = we1_ref[0]
    w2m = we2_ref[0]
    b1 = be1_ref[0]
    b2 = be2_ref[0]
    d1 = d1_ref[...]
    d2 = d2_ref[...]
    gv1 = w1_ref[...]
    gv2 = w2_ref[...]
    limit = (off + cnt).astype(jnp.float32)
    n_tiles = (cnt + MOE_TILE - 1) // MOE_TILE

    def body(i, carry):
        p0 = (off + i * MOE_TILE).astype(jnp.float32)
        prow = jax.lax.broadcasted_iota(
            jnp.int32, (1, MOE_TILE), 1).astype(jnp.float32) + p0
        valid = prow < limit
        ma = ((d1 == prow) & valid).astype(jnp.float32)
        mb = ((d2 == prow) & valid).astype(jnp.float32)
        m = ma + mb
        xt = jax.lax.dot_general(m, x2_ref[...], (((0,), (0,)), ((), ())),
                                 preferred_element_type=jnp.float32)
        h1 = _silu(jnp.dot(xt, w1m, preferred_element_type=jnp.float32) + b1)
        h2 = jnp.dot(h1, w2m, preferred_element_type=jnp.float32) + b2
        mw = ma * gv1 + mb * gv2
        out_ref[...] += jnp.dot(mw, h2, preferred_element_type=jnp.float32)
        return carry

    jax.lax.fori_loop(0, n_tiles, body, 0)


def _rope_full():
    freqs = 1.0 / (THETA ** (jnp.arange(0, HD, 2)[: HD // 2]
                             .astype(jnp.float32) / HD))
    t = jnp.arange(T, dtype=jnp.float32)
    f = jnp.outer(t, freqs)
    cos = jnp.repeat(jnp.cos(f), 2, axis=1)
    sin = jnp.repeat(jnp.sin(f), 2, axis=1)
    return jnp.tile(cos, (1, H)), jnp.tile(sin, (1, H))


def _rot_cols(w):
    r = w.reshape(*w.shape[:-1], w.shape[-1] // 2, 2)
    return jnp.stack([-r[..., 1], r[..., 0]], axis=-1).reshape(w.shape)


def kernel(x, ln1_w, ln2_w, Wqkv, bqkv, Wproj, bproj, gate_w, gate_b,
           We1, be1, We2, be2, Ws1, bs1, Ws2, bs2):
    xf = x.reshape(T, C)
    cosE, sinE = _rope_full()
    Wq, Wk = Wqkv[:, :C], Wqkv[:, C:2 * C]
    W_ext = jnp.concatenate([Wqkv, _rot_cols(Wq), _rot_cols(Wk)], axis=1)
    b_ext = jnp.concatenate([bqkv, _rot_cols(bqkv[:C]), _rot_cols(bqkv[C:2 * C])])

    f32 = jnp.float32
    nq = T // QKV_ROWS
    q, k, v = pl.pallas_call(
        _qkv_kernel,
        grid=(nq,),
        in_specs=[
            pl.BlockSpec((QKV_ROWS, C), lambda i: (i, 0)),
            pl.BlockSpec((1, C), lambda i: (0, 0)),
            pl.BlockSpec((C, 5 * C), lambda i: (0, 0)),
            pl.BlockSpec((1, 5 * C), lambda i: (0, 0)),
            pl.BlockSpec((QKV_ROWS, C), lambda i: (i, 0)),
            pl.BlockSpec((QKV_ROWS, C), lambda i: (i, 0)),
        ],
        out_specs=[pl.BlockSpec((QKV_ROWS, C), lambda i: (i, 0))] * 3,
        out_shape=[jax.ShapeDtypeStruct((T, C), f32)] * 3,
    )(xf, ln1_w.reshape(1, C), W_ext, b_ext.reshape(1, 5 * C), cosE, sinE)

    qh = q.reshape(T, H, HD).transpose(1, 0, 2)
    kh = k.reshape(T, H, HD).transpose(1, 0, 2)
    vh = v.reshape(T, H, HD).transpose(1, 0, 2)
    na = T // ATT_ROWS
    yh = pl.pallas_call(
        _att_kernel,
        grid=(H, na),
        in_specs=[
            pl.BlockSpec((1, ATT_ROWS, HD), lambda h, i: (h, i, 0)),
            pl.BlockSpec((1, T, HD), lambda h, i: (h, 0, 0)),
            pl.BlockSpec((1, T, HD), lambda h, i: (h, 0, 0)),
        ],
        out_specs=pl.BlockSpec((1, ATT_ROWS, HD), lambda h, i: (h, i, 0)),
        out_shape=jax.ShapeDtypeStruct((H, T, HD), f32),
    )(qh, kh, vh)
    y = yh.transpose(1, 0, 2).reshape(T, C)

    x1, x2, d1, d2, gw1, gw2, offs, counts, loss = pl.pallas_call(
        _post_kernel,
        grid=(1,),
        in_specs=[
            pl.BlockSpec((T, C), lambda i: (0, 0)),
            pl.BlockSpec((T, C), lambda i: (0, 0)),
            pl.BlockSpec((C, C), lambda i: (0, 0)),
            pl.BlockSpec((1, C), lambda i: (0, 0)),
            pl.BlockSpec((1, C), lambda i: (0, 0)),
            pl.BlockSpec((C, E), lambda i: (0, 0)),
            pl.BlockSpec((1, E), lambda i: (0, 0)),
        ],
        out_specs=[
            pl.BlockSpec((T, C), lambda i: (0, 0)),
            pl.BlockSpec((T, C), lambda i: (0, 0)),
            pl.BlockSpec((T, 1), lambda i: (0, 0)),
            pl.BlockSpec((T, 1), lambda i: (0, 0)),
            pl.BlockSpec((T, 1), lambda i: (0, 0)),
            pl.BlockSpec((T, 1), lambda i: (0, 0)),
            pl.BlockSpec((1, E), lambda i: (0, 0)),
            pl.BlockSpec((1, E), lambda i: (0, 0)),
            pl.BlockSpec((1, 1), lambda i: (0, 0)),
        ],
        out_shape=[
            jax.ShapeDtypeStruct((T, C), f32),
            jax.ShapeDtypeStruct((T, C), f32),
            jax.ShapeDtypeStruct((T, 1), f32),
            jax.ShapeDtypeStruct((T, 1), f32),
            jax.ShapeDtypeStruct((T, 1), f32),
            jax.ShapeDtypeStruct((T, 1), f32),
            jax.ShapeDtypeStruct((1, E), f32),
            jax.ShapeDtypeStruct((1, E), f32),
            jax.ShapeDtypeStruct((1, 1), f32),
        ],
    )(xf, y, Wproj, bproj.reshape(1, C), ln2_w.reshape(1, C),
      gate_w, gate_b.reshape(1, E))

    ns = T // SH_ROWS
    base = pl.pallas_call(
        _shared_kernel,
        grid=(ns,),
        in_specs=[
            pl.BlockSpec((SH_ROWS, C), lambda i: (i, 0)),
            pl.BlockSpec((SH_ROWS, C), lambda i: (i, 0)),
            pl.BlockSpec((C, DFF), lambda i: (0, 0)),
            pl.BlockSpec((1, DFF), lambda i: (0, 0)),
            pl.BlockSpec((DFF, C), lambda i: (0, 0)),
            pl.BlockSpec((1, C), lambda i: (0, 0)),
        ],
        out_specs=pl.BlockSpec((SH_ROWS, C), lambda i: (i, 0)),
        out_shape=jax.ShapeDtypeStruct((T, C), f32),
    )(x1, x2, Ws1, bs1.reshape(1, DFF), Ws2, bs2.reshape(1, C))

    offcnt = jnp.concatenate([offs, counts], axis=1).reshape(2 * E) \
        .astype(jnp.int32)

    out = pl.pallas_call(
        _moe_kernel,
        grid_spec=pltpu.PrefetchScalarGridSpec(
            num_scalar_prefetch=1,
            grid=(E,),
            in_specs=[
                pl.BlockSpec((T, C), lambda e, s: (0, 0)),
                pl.BlockSpec((T, 1), lambda e, s: (0, 0)),
                pl.BlockSpec((T, 1), lambda e, s: (0, 0)),
                pl.BlockSpec((T, 1), lambda e, s: (0, 0)),
                pl.BlockSpec((T, 1), lambda e, s: (0, 0)),
                pl.BlockSpec((T, C), lambda e, s: (0, 0)),
                pl.BlockSpec((1, C, DFF), lambda e, s: (e, 0, 0)),
                pl.BlockSpec((1, 1, DFF), lambda e, s: (e, 0, 0)),
                pl.BlockSpec((1, DFF, C), lambda e, s: (e, 0, 0)),
                pl.BlockSpec((1, 1, C), lambda e, s: (e, 0, 0)),
            ],
            out_specs=pl.BlockSpec((T, C), lambda e, s: (0, 0)),
        ),
        out_shape=jax.ShapeDtypeStruct((T, C), f32),
    )(offcnt, x2, d1, d2, gw1, gw2, base, We1, be1.reshape(E, 1, DFF),
      We2, be2.reshape(E, 1, C))

    return out.reshape(B, T, C), loss.reshape(())


# speedup vs baseline: 3.4148x; 1.0700x over previous
import functools

import jax
import jax.numpy as jnp
from jax.experimental import pallas as pl
from jax.experimental.pallas import tpu as pltpu

B, T, C = 1, 2048, 768
H = 12
HD = C // H
E = 64
TOPK = 2
DFF = 2 * C
THETA = 10000.0
ALPHA = 0.01
EPS = 1e-5

QKV_ROWS = 512
ATT_ROWS = 512
MOE_TILE = 128
SH_ROWS = 512


def _rms(x, w):
    return x * jax.lax.rsqrt(jnp.mean(x * x, axis=-1, keepdims=True) + EPS) * w


def _silu(a):
    return a * (1.0 / (1.0 + jnp.exp(-a)))


def _qkv_kernel(x_ref, ln1_ref, wext_ref, bext_ref, cos_ref, sin_ref,
                q_ref, k_ref, v_ref):
    h = _rms(x_ref[...], ln1_ref[...]).astype(jnp.bfloat16)
    z = jnp.dot(h, wext_ref[...], preferred_element_type=jnp.float32)
    z = z + bext_ref[...]
    q = z[:, :C]
    k = z[:, C:2 * C]
    v = z[:, 2 * C:3 * C]
    qs = z[:, 3 * C:4 * C]
    ks = z[:, 4 * C:5 * C]
    cos = cos_ref[...]
    sin = sin_ref[...]
    q_ref[...] = (q * cos + qs * sin).astype(jnp.bfloat16)
    k_ref[...] = (k * cos + ks * sin).astype(jnp.bfloat16)
    v_ref[...] = v.astype(jnp.bfloat16)


def _att_kernel(q_ref, k_ref, v_ref, o_ref):
    i = pl.program_id(1)
    q = q_ref[0]
    s = jax.lax.dot_general(q, k_ref[0], (((1,), (1,)), ((), ())),
                            preferred_element_type=jnp.float32)
    s = s * (1.0 / (HD ** 0.5))
    row = jax.lax.broadcasted_iota(jnp.int32, (ATT_ROWS, T), 0) + i * ATT_ROWS
    col = jax.lax.broadcasted_iota(jnp.int32, (ATT_ROWS, T), 1)
    s = jnp.where(row >= col, s, jnp.float32(-1e30))
    m = jnp.max(s, axis=-1, keepdims=True)
    p = jnp.exp(s - m)
    p = (p / jnp.sum(p, axis=-1, keepdims=True)).astype(jnp.bfloat16)
    o_ref[0] = jnp.dot(p, v_ref[0],
                       preferred_element_type=jnp.float32).astype(jnp.bfloat16)


def _post_kernel(x_ref, y_ref, wp_ref, bp_ref, ln2_ref, gw_ref, gb_ref,
                 x1_ref, x2_ref, x2b_ref, d1_ref, d2_ref, w1_ref, w2_ref,
                 off_ref, cnt_ref, loss_ref):
    x1 = x_ref[...] + jnp.dot(y_ref[...], wp_ref[...],
                              preferred_element_type=jnp.float32) + bp_ref[...]
    x1_ref[...] = x1
    x2 = _rms(x1, ln2_ref[...])
    x2_ref[...] = x2
    x2b_ref[...] = x2.astype(jnp.bfloat16)
    logits = jnp.dot(x2, gw_ref[...], preferred_element_type=jnp.float32)
    lmax = jnp.max(logits, axis=-1, keepdims=True)
    ex = jnp.exp(logits - lmax)
    a = ex / jnp.sum(ex, axis=-1, keepdims=True)
    b = a + gb_ref[...]
    lane = jax.lax.broadcasted_iota(jnp.int32, (T, E), 1)
    m1 = jnp.max(b, axis=-1, keepdims=True)
    i1 = jnp.min(jnp.where(b == m1, lane, E), axis=-1, keepdims=True)
    oh1 = (lane == i1).astype(jnp.float32)
    b2 = jnp.where(lane == i1, jnp.float32(-1e30), b)
    m2 = jnp.max(b2, axis=-1, keepdims=True)
    i2 = jnp.min(jnp.where(b2 == m2, lane, E), axis=-1, keepdims=True)
    oh2 = (lane == i2).astype(jnp.float32)
    w1_ref[...] = jnp.sum(oh1 * a, axis=-1, keepdims=True)
    w2_ref[...] = jnp.sum(oh2 * a, axis=-1, keepdims=True)
    ohs = oh1 + oh2
    counts = jnp.sum(ohs, axis=0, keepdims=True)
    rt = jax.lax.broadcasted_iota(jnp.int32, (T, T), 0)
    ct = jax.lax.broadcasted_iota(jnp.int32, (T, T), 1)
    ltri = (rt > ct).astype(jnp.bfloat16)
    cum = jnp.dot(ltri, ohs.astype(jnp.bfloat16),
                  preferred_element_type=jnp.float32)
    re = jax.lax.broadcasted_iota(jnp.int32, (E, E), 0)
    ce = jax.lax.broadcasted_iota(jnp.int32, (E, E), 1)
    upe = (re < ce).astype(jnp.float32)
    offs = jnp.dot(counts, upe, preferred_element_type=jnp.float32)
    base = cum + offs
    d1_ref[...] = jnp.sum(oh1 * base, axis=-1, keepdims=True)
    d2_ref[...] = jnp.sum(oh2 * base, axis=-1, keepdims=True) + \
        jnp.sum(oh1 * oh2, axis=-1, keepdims=True)
    off_ref[...] = offs
    cnt_ref[...] = counts
    probs = jnp.mean(a, axis=0, keepdims=True)
    f_i = counts * jnp.float32(E) / (jnp.float32(TOPK * T) + 1e-6)
    loss_ref[...] = jnp.sum(f_i * probs, keepdims=True).reshape(1, 1) * ALPHA


def _shared_kernel(x1_ref, x2_ref, ws1_ref, bs1_ref, ws2_ref, bs2_ref,
                   o_ref):
    h = _silu(jnp.dot(x2_ref[...], ws1_ref[...],
                      preferred_element_type=jnp.float32) + bs1_ref[...])
    o_ref[...] = x1_ref[...] + jnp.dot(h.astype(jnp.bfloat16), ws2_ref[...],
                                       preferred_element_type=jnp.float32) \
        + bs2_ref[...]


def _moe_kernel(sc_ref, x2_ref, d1_ref, d2_ref, w1_ref, w2_ref, base_ref,
                we1_ref, be1_ref, we2_ref, be2_ref, out_ref,
                w1s_ref, w2s_ref):
    e = pl.program_id(0)
    off = sc_ref[e]
    cnt = sc_ref[E + e]

    @pl.when(e == 0)
    def _():
        out_ref[...] = base_ref[...]

    w1s_ref[...] = we1_ref[0].astype(jnp.bfloat16)
    w2s_ref[...] = we2_ref[0].astype(jnp.bfloat16)
    w1m = w1s_ref[...]
    w2m = w2s_ref[...]
    b1 = be1_ref[0]
    b2 = be2_ref[0]
    d1 = d1_ref[...]
    d2 = d2_ref[...]
    gv1 = w1_ref[...].astype(jnp.bfloat16)
    gv2 = w2_ref[...].astype(jnp.bfloat16)
    limit = (off + cnt).astype(jnp.float32)
    n_tiles = (cnt + MOE_TILE - 1) // MOE_TILE

    def body(i, carry):
        p0 = (off + i * MOE_TILE).astype(jnp.float32)
        prow = jax.lax.broadcasted_iota(
            jnp.int32, (1, MOE_TILE), 1).astype(jnp.float32) + p0
        valid = prow < limit
        ma = ((d1 == prow) & valid).astype(jnp.bfloat16)
        mb = ((d2 == prow) & valid).astype(jnp.bfloat16)
        m = ma + mb
        xt = jax.lax.dot_general(m, x2_ref[...], (((0,), (0,)), ((), ())),
                                 preferred_element_type=jnp.float32)
        h1 = _silu(jnp.dot(xt.astype(jnp.bfloat16), w1m,
                           preferred_element_type=jnp.float32) + b1)
        h2 = jnp.dot(h1.astype(jnp.bfloat16), w2m,
                     preferred_element_type=jnp.float32) + b2
        mw = ma * gv1 + mb * gv2
        out_ref[...] += jnp.dot(mw, h2.astype(jnp.bfloat16),
                                preferred_element_type=jnp.float32)
        return carry

    jax.lax.fori_loop(0, n_tiles, body, 0)


def _rope_full():
    freqs = 1.0 / (THETA ** (jnp.arange(0, HD, 2)[: HD // 2]
                             .astype(jnp.float32) / HD))
    t = jnp.arange(T, dtype=jnp.float32)
    f = jnp.outer(t, freqs)
    cos = jnp.repeat(jnp.cos(f), 2, axis=1)
    sin = jnp.repeat(jnp.sin(f), 2, axis=1)
    return jnp.tile(cos, (1, H)), jnp.tile(sin, (1, H))


def _rot_cols(w):
    r = w.reshape(*w.shape[:-1], w.shape[-1] // 2, 2)
    return jnp.stack([-r[..., 1], r[..., 0]], axis=-1).reshape(w.shape)


def kernel(x, ln1_w, ln2_w, Wqkv, bqkv, Wproj, bproj, gate_w, gate_b,
           We1, be1, We2, be2, Ws1, bs1, Ws2, bs2):
    xf = x.reshape(T, C)
    cosE, sinE = _rope_full()
    Wq, Wk = Wqkv[:, :C], Wqkv[:, C:2 * C]
    W_ext = jnp.concatenate([Wqkv, _rot_cols(Wq), _rot_cols(Wk)],
                            axis=1).astype(jnp.bfloat16)
    b_ext = jnp.concatenate([bqkv, _rot_cols(bqkv[:C]), _rot_cols(bqkv[C:2 * C])])

    f32 = jnp.float32
    nq = T // QKV_ROWS
    q, k, v = pl.pallas_call(
        _qkv_kernel,
        grid=(nq,),
        in_specs=[
            pl.BlockSpec((QKV_ROWS, C), lambda i: (i, 0)),
            pl.BlockSpec((1, C), lambda i: (0, 0)),
            pl.BlockSpec((C, 5 * C), lambda i: (0, 0)),
            pl.BlockSpec((1, 5 * C), lambda i: (0, 0)),
            pl.BlockSpec((QKV_ROWS, C), lambda i: (i, 0)),
            pl.BlockSpec((QKV_ROWS, C), lambda i: (i, 0)),
        ],
        out_specs=[pl.BlockSpec((QKV_ROWS, C), lambda i: (i, 0))] * 3,
        out_shape=[jax.ShapeDtypeStruct((T, C), jnp.bfloat16)] * 3,
    )(xf, ln1_w.reshape(1, C), W_ext, b_ext.reshape(1, 5 * C), cosE, sinE)

    qh = q.reshape(T, H, HD).transpose(1, 0, 2)
    kh = k.reshape(T, H, HD).transpose(1, 0, 2)
    vh = v.reshape(T, H, HD).transpose(1, 0, 2)
    na = T // ATT_ROWS
    yh = pl.pallas_call(
        _att_kernel,
        grid=(H, na),
        in_specs=[
            pl.BlockSpec((1, ATT_ROWS, HD), lambda h, i: (h, i, 0)),
            pl.BlockSpec((1, T, HD), lambda h, i: (h, 0, 0)),
            pl.BlockSpec((1, T, HD), lambda h, i: (h, 0, 0)),
        ],
        out_specs=pl.BlockSpec((1, ATT_ROWS, HD), lambda h, i: (h, i, 0)),
        out_shape=jax.ShapeDtypeStruct((H, T, HD), jnp.bfloat16),
    )(qh, kh, vh)
    y = yh.transpose(1, 0, 2).reshape(T, C)

    x1, x2, x2b, d1, d2, gw1, gw2, offs, counts, loss = pl.pallas_call(
        _post_kernel,
        grid=(1,),
        in_specs=[
            pl.BlockSpec((T, C), lambda i: (0, 0)),
            pl.BlockSpec((T, C), lambda i: (0, 0)),
            pl.BlockSpec((C, C), lambda i: (0, 0)),
            pl.BlockSpec((1, C), lambda i: (0, 0)),
            pl.BlockSpec((1, C), lambda i: (0, 0)),
            pl.BlockSpec((C, E), lambda i: (0, 0)),
            pl.BlockSpec((1, E), lambda i: (0, 0)),
        ],
        out_specs=[
            pl.BlockSpec((T, C), lambda i: (0, 0)),
            pl.BlockSpec((T, C), lambda i: (0, 0)),
            pl.BlockSpec((T, C), lambda i: (0, 0)),
            pl.BlockSpec((T, 1), lambda i: (0, 0)),
            pl.BlockSpec((T, 1), lambda i: (0, 0)),
            pl.BlockSpec((T, 1), lambda i: (0, 0)),
            pl.BlockSpec((T, 1), lambda i: (0, 0)),
            pl.BlockSpec((1, E), lambda i: (0, 0)),
            pl.BlockSpec((1, E), lambda i: (0, 0)),
            pl.BlockSpec((1, 1), lambda i: (0, 0)),
        ],
        out_shape=[
            jax.ShapeDtypeStruct((T, C), f32),
            jax.ShapeDtypeStruct((T, C), f32),
            jax.ShapeDtypeStruct((T, C), jnp.bfloat16),
            jax.ShapeDtypeStruct((T, 1), f32),
            jax.ShapeDtypeStruct((T, 1), f32),
            jax.ShapeDtypeStruct((T, 1), f32),
            jax.ShapeDtypeStruct((T, 1), f32),
            jax.ShapeDtypeStruct((1, E), f32),
            jax.ShapeDtypeStruct((1, E), f32),
            jax.ShapeDtypeStruct((1, 1), f32),
        ],
    )(xf, y, Wproj.astype(jnp.bfloat16), bproj.reshape(1, C),
      ln2_w.reshape(1, C), gate_w, gate_b.reshape(1, E))

    ns = T // SH_ROWS
    base = pl.pallas_call(
        _shared_kernel,
        grid=(ns,),
        in_specs=[
            pl.BlockSpec((SH_ROWS, C), lambda i: (i, 0)),
            pl.BlockSpec((SH_ROWS, C), lambda i: (i, 0)),
            pl.BlockSpec((C, DFF), lambda i: (0, 0)),
            pl.BlockSpec((1, DFF), lambda i: (0, 0)),
            pl.BlockSpec((DFF, C), lambda i: (0, 0)),
            pl.BlockSpec((1, C), lambda i: (0, 0)),
        ],
        out_specs=pl.BlockSpec((SH_ROWS, C), lambda i: (i, 0)),
        out_shape=jax.ShapeDtypeStruct((T, C), f32),
    )(x1, x2b, Ws1.astype(jnp.bfloat16), bs1.reshape(1, DFF),
      Ws2.astype(jnp.bfloat16), bs2.reshape(1, C))

    offcnt = jnp.concatenate([offs, counts], axis=1).reshape(2 * E) \
        .astype(jnp.int32)

    out = pl.pallas_call(
        _moe_kernel,
        grid_spec=pltpu.PrefetchScalarGridSpec(
            num_scalar_prefetch=1,
            grid=(E,),
            in_specs=[
                pl.BlockSpec((T, C), lambda e, s: (0, 0)),
                pl.BlockSpec((T, 1), lambda e, s: (0, 0)),
                pl.BlockSpec((T, 1), lambda e, s: (0, 0)),
                pl.BlockSpec((T, 1), lambda e, s: (0, 0)),
                pl.BlockSpec((T, 1), lambda e, s: (0, 0)),
                pl.BlockSpec((T, C), lambda e, s: (0, 0)),
                pl.BlockSpec((1, C, DFF), lambda e, s: (e, 0, 0)),
                pl.BlockSpec((1, 1, DFF), lambda e, s: (e, 0, 0)),
                pl.BlockSpec((1, DFF, C), lambda e, s: (e, 0, 0)),
                pl.BlockSpec((1, 1, C), lambda e, s: (e, 0, 0)),
            ],
            out_specs=pl.BlockSpec((T, C), lambda e, s: (0, 0)),
            scratch_shapes=[
                pltpu.VMEM((C, DFF), jnp.bfloat16),
                pltpu.VMEM((DFF, C), jnp.bfloat16),
            ],
        ),
        out_shape=jax.ShapeDtypeStruct((T, C), f32),
    )(offcnt, x2b, d1, d2, gw1, gw2, base, We1, be1.reshape(E, 1, DFF),
      We2, be2.reshape(E, 1, C))

    return out.reshape(B, T, C), loss.reshape(())


# speedup vs baseline: 3.4638x; 1.0144x over previous
import jax
import jax.numpy as jnp
from jax.experimental import pallas as pl
from jax.experimental.pallas import tpu as pltpu

B, T, C = 1, 2048, 768
H = 12
HD = C // H
E = 64
TOPK = 2
DFF = 2 * C
THETA = 10000.0
ALPHA = 0.01
EPS = 1e-5

QKV_ROWS = 512
ATT_ROWS = 512
MOE_TILE = 128
SH_ROWS = 512
NS = TOPK * T + E * 7 + MOE_TILE
SCHUNK = NS // 4

BF = jnp.bfloat16
F32 = jnp.float32


def _rms(x, w):
    return x * jax.lax.rsqrt(jnp.mean(x * x, axis=-1, keepdims=True) + EPS) * w


def _silu(a):
    return a * (1.0 / (1.0 + jnp.exp(-a)))


def _qkv_kernel(x_ref, ln1_ref, wext_ref, bext_ref, cos_ref, sin_ref,
                q_ref, k_ref, v_ref):
    h = _rms(x_ref[...], ln1_ref[...]).astype(BF)
    z = jnp.dot(h, wext_ref[...], preferred_element_type=F32)
    z = z + bext_ref[...]
    q = z[:, :C]
    k = z[:, C:2 * C]
    v = z[:, 2 * C:3 * C]
    qs = z[:, 3 * C:4 * C]
    ks = z[:, 4 * C:5 * C]
    cos = cos_ref[...]
    sin = sin_ref[...]
    qr = (q * cos + qs * sin).astype(BF)
    kr = (k * cos + ks * sin).astype(BF)
    vb = v.astype(BF)
    for hh in range(H):
        sl = slice(hh * HD, (hh + 1) * HD)
        q_ref[hh] = qr[:, sl]
        k_ref[hh] = kr[:, sl]
        v_ref[hh] = vb[:, sl]


def _att_kernel(q_ref, k_ref, v_ref, o_ref, m_ref, l_ref, acc_ref):
    i = pl.program_id(1)
    j = pl.program_id(2)

    @pl.when(j <= i)
    def _():
        s = jax.lax.dot_general(q_ref[0], k_ref[0], (((1,), (1,)), ((), ())),
                                preferred_element_type=F32)
        s = s * (1.0 / (HD ** 0.5))
        row = jax.lax.broadcasted_iota(jnp.int32, (ATT_ROWS, ATT_ROWS), 0) \
            + i * ATT_ROWS
        col = jax.lax.broadcasted_iota(jnp.int32, (ATT_ROWS, ATT_ROWS), 1) \
            + j * ATT_ROWS
        s = jnp.where(row >= col, s, jnp.float32(-1e30))
        mt = jnp.max(s, axis=-1, keepdims=True)

        @pl.when(j == 0)
        def _():
            p = jnp.exp(s - mt)
            m_ref[...] = mt
            l_ref[...] = jnp.sum(p, axis=-1, keepdims=True)
            acc_ref[...] = jnp.dot(p.astype(BF), v_ref[0],
                                   preferred_element_type=F32)

        @pl.when(j > 0)
        def _():
            m_old = m_ref[...]
            m_new = jnp.maximum(m_old, mt)
            corr = jnp.exp(m_old - m_new)
            p = jnp.exp(s - m_new)
            m_ref[...] = m_new
            l_ref[...] = l_ref[...] * corr + jnp.sum(p, axis=-1, keepdims=True)
            acc_ref[...] = acc_ref[...] * corr + \
                jnp.dot(p.astype(BF), v_ref[0], preferred_element_type=F32)

        @pl.when(j == i)
        def _():
            o_ref[0] = (acc_ref[...] / l_ref[...]).astype(BF)


def _post_kernel(x_ref, yh_ref, wp_ref, bp_ref, ln2_ref, gw_ref, gb_ref,
                 x1_ref, x2b_ref, d1_ref, d2_ref, w1_ref, w2_ref,
                 off_ref, cnt_ref, loss_ref):
    acc = bp_ref[...] + jnp.zeros((T, C), F32)
    for hh in range(H):
        acc = acc + jnp.dot(yh_ref[hh], wp_ref[hh * HD:(hh + 1) * HD, :],
                            preferred_element_type=F32)
    x1 = x_ref[...] + acc
    x1_ref[...] = x1
    x2 = _rms(x1, ln2_ref[...])
    x2b_ref[...] = x2.astype(BF)
    logits = jnp.dot(x2, gw_ref[...], preferred_element_type=F32)
    lmax = jnp.max(logits, axis=-1, keepdims=True)
    ex = jnp.exp(logits - lmax)
    a = ex / jnp.sum(ex, axis=-1, keepdims=True)
    b = a + gb_ref[...]
    lane = jax.lax.broadcasted_iota(jnp.int32, (T, E), 1)
    m1 = jnp.max(b, axis=-1, keepdims=True)
    i1 = jnp.min(jnp.where(b == m1, lane, E), axis=-1, keepdims=True)
    oh1 = (lane == i1).astype(F32)
    b2 = jnp.where(lane == i1, jnp.float32(-1e30), b)
    m2 = jnp.max(b2, axis=-1, keepdims=True)
    i2 = jnp.min(jnp.where(b2 == m2, lane, E), axis=-1, keepdims=True)
    oh2 = (lane == i2).astype(F32)
    w1_ref[...] = jnp.sum(oh1 * a, axis=-1, keepdims=True)
    w2_ref[...] = jnp.sum(oh2 * a, axis=-1, keepdims=True)
    ohs = oh1 + oh2
    counts = jnp.sum(ohs, axis=0, keepdims=True)
    rt = jax.lax.broadcasted_iota(jnp.int32, (T, T), 0)
    ct = jax.lax.broadcasted_iota(jnp.int32, (T, T), 1)
    ltri = (rt > ct).astype(BF)
    cum = jnp.dot(ltri, ohs.astype(BF), preferred_element_type=F32)
    re = jax.lax.broadcasted_iota(jnp.int32, (E, E), 0)
    ce = jax.lax.broadcasted_iota(jnp.int32, (E, E), 1)
    upe = (re < ce).astype(F32)
    counts8 = jnp.floor((counts + 7.0) * 0.125) * 8.0
    offs = jnp.dot(counts8, upe, preferred_element_type=F32)
    base = cum + offs
    d1_ref[...] = jnp.sum(oh1 * base, axis=-1, keepdims=True)
    d2_ref[...] = jnp.sum(oh2 * base, axis=-1, keepdims=True) + \
        jnp.sum(oh1 * oh2, axis=-1, keepdims=True)
    off_ref[...] = offs * 0.125
    cnt_ref[...] = counts
    probs = jnp.mean(a, axis=0, keepdims=True)
    f_i = counts * jnp.float32(E) / (jnp.float32(TOPK * T) + 1e-6)
    loss_ref[...] = jnp.sum(f_i * probs, keepdims=True).reshape(1, 1) * ALPHA


def _shared_kernel(x1_ref, x2_ref, ws1_ref, bs1_ref, ws2_ref, bs2_ref,
                   o_ref):
    h = _silu(jnp.dot(x2_ref[...], ws1_ref[...],
                      preferred_element_type=F32) + bs1_ref[...])
    o_ref[...] = x1_ref[...] + jnp.dot(h.astype(BF), ws2_ref[...],
                                       preferred_element_type=F32) \
        + bs2_ref[...]


def _moe_kernel(sc_ref, x2_ref, d1_ref, d2_ref, w1_ref, w2_ref,
                we1_ref, be1_ref, we2_ref, be2_ref, hs_ref):
    e = pl.program_id(0)
    soff = sc_ref[e]
    cnt = sc_ref[E + e]
    off = soff * 8

    @pl.when(e == 0)
    def _():
        hs_ref[...] = jnp.zeros((NS, C), BF)

    w1m = we1_ref[0]
    w2m = we2_ref[0]
    b1 = be1_ref[0]
    b2 = be2_ref[0]
    d1 = d1_ref[...]
    d2 = d2_ref[...]
    gv1 = w1_ref[...].astype(BF)
    gv2 = w2_ref[...].astype(BF)
    limit = (off + cnt).astype(F32)
    n_tiles = (cnt + MOE_TILE - 1) // MOE_TILE

    def body(i, carry):
        p0 = soff * 8 + i * MOE_TILE
        p0f = p0.astype(F32)
        prow = jax.lax.broadcasted_iota(
            jnp.int32, (1, MOE_TILE), 1).astype(F32) + p0f
        valid = prow < limit
        ma = ((d1 == prow) & valid).astype(BF)
        mb = ((d2 == prow) & valid).astype(BF)
        m = ma + mb
        xt = jax.lax.dot_general(m, x2_ref[...], (((0,), (0,)), ((), ())),
                                 preferred_element_type=F32)
        gw = jax.lax.dot_general(ma, gv1, (((0,), (0,)), ((), ())),
                                 preferred_element_type=F32) + \
            jax.lax.dot_general(mb, gv2, (((0,), (0,)), ((), ())),
                                preferred_element_type=F32)
        h1 = _silu(jnp.dot(xt, w1m, preferred_element_type=F32) + b1)
        h2 = jnp.dot(h1, w2m, preferred_element_type=F32) + b2
        hs_ref[pl.ds(p0, MOE_TILE), :] = (h2 * gw).astype(BF)
        return carry

    jax.lax.fori_loop(0, n_tiles, body, 0)


def _scatter_kernel(d1_ref, d2_ref, hs_ref, base_ref, out_ref):
    c = pl.program_id(0)
    prow = jax.lax.broadcasted_iota(
        jnp.int32, (1, SCHUNK), 1).astype(F32) + (c * SCHUNK).astype(F32)
    d1 = d1_ref[...]
    d2 = d2_ref[...]
    m = (d1 == prow).astype(BF) + (d2 == prow).astype(BF)
    contrib = jnp.dot(m, hs_ref[...], preferred_element_type=F32)

    @pl.when(c == 0)
    def _():
        out_ref[...] = base_ref[...] + contrib

    @pl.when(c > 0)
    def _():
        out_ref[...] += contrib


def _rope_full():
    freqs = 1.0 / (THETA ** (jnp.arange(0, HD, 2)[: HD // 2]
                             .astype(F32) / HD))
    t = jnp.arange(T, dtype=F32)
    f = jnp.outer(t, freqs)
    cos = jnp.repeat(jnp.cos(f), 2, axis=1)
    sin = jnp.repeat(jnp.sin(f), 2, axis=1)
    return jnp.tile(cos, (1, H)), jnp.tile(sin, (1, H))


def _rot_cols(w):
    r = w.reshape(*w.shape[:-1], w.shape[-1] // 2, 2)
    return jnp.stack([-r[..., 1], r[..., 0]], axis=-1).reshape(w.shape)


def kernel(x, ln1_w, ln2_w, Wqkv, bqkv, Wproj, bproj, gate_w, gate_b,
           We1, be1, We2, be2, Ws1, bs1, Ws2, bs2):
    xf = x.reshape(T, C)
    cosE, sinE = _rope_full()
    Wq, Wk = Wqkv[:, :C], Wqkv[:, C:2 * C]
    W_ext = jnp.concatenate([Wqkv, _rot_cols(Wq), _rot_cols(Wk)],
                            axis=1).astype(BF)
    b_ext = jnp.concatenate([bqkv, _rot_cols(bqkv[:C]),
                             _rot_cols(bqkv[C:2 * C])])

    nq = T // QKV_ROWS
    q, k, v = pl.pallas_call(
        _qkv_kernel,
        grid=(nq,),
        in_specs=[
            pl.BlockSpec((QKV_ROWS, C), lambda i: (i, 0)),
            pl.BlockSpec((1, C), lambda i: (0, 0)),
            pl.BlockSpec((C, 5 * C), lambda i: (0, 0)),
            pl.BlockSpec((1, 5 * C), lambda i: (0, 0)),
            pl.BlockSpec((QKV_ROWS, C), lambda i: (i, 0)),
            pl.BlockSpec((QKV_ROWS, C), lambda i: (i, 0)),
        ],
        out_specs=[pl.BlockSpec((H, QKV_ROWS, HD), lambda i: (0, i, 0))] * 3,
        out_shape=[jax.ShapeDtypeStruct((H, T, HD), BF)] * 3,
    )(xf, ln1_w.reshape(1, C), W_ext, b_ext.reshape(1, 5 * C), cosE, sinE)

    na = T // ATT_ROWS
    yh = pl.pallas_call(
        _att_kernel,
        grid=(H, na, na),
        in_specs=[
            pl.BlockSpec((1, ATT_ROWS, HD), lambda h, i, j: (h, i, 0)),
            pl.BlockSpec((1, ATT_ROWS, HD), lambda h, i, j: (h, j, 0)),
            pl.BlockSpec((1, ATT_ROWS, HD), lambda h, i, j: (h, j, 0)),
        ],
        out_specs=pl.BlockSpec((1, ATT_ROWS, HD), lambda h, i, j: (h, i, 0)),
        out_shape=jax.ShapeDtypeStruct((H, T, HD), BF),
        scratch_shapes=[
            pltpu.VMEM((ATT_ROWS, 1), F32),
            pltpu.VMEM((ATT_ROWS, 1), F32),
            pltpu.VMEM((ATT_ROWS, HD), F32),
        ],
    )(q, k, v)

    x1, x2b, d1, d2, gw1, gw2, offs, counts, loss = pl.pallas_call(
        _post_kernel,
        grid=(1,),
        in_specs=[
            pl.BlockSpec((T, C), lambda i: (0, 0)),
            pl.BlockSpec((H, T, HD), lambda i: (0, 0, 0)),
            pl.BlockSpec((C, C), lambda i: (0, 0)),
            pl.BlockSpec((1, C), lambda i: (0, 0)),
            pl.BlockSpec((1, C), lambda i: (0, 0)),
            pl.BlockSpec((C, E), lambda i: (0, 0)),
            pl.BlockSpec((1, E), lambda i: (0, 0)),
        ],
        out_specs=[
            pl.BlockSpec((T, C), lambda i: (0, 0)),
            pl.BlockSpec((T, C), lambda i: (0, 0)),
            pl.BlockSpec((T, 1), lambda i: (0, 0)),
            pl.BlockSpec((T, 1), lambda i: (0, 0)),
            pl.BlockSpec((T, 1), lambda i: (0, 0)),
            pl.BlockSpec((T, 1), lambda i: (0, 0)),
            pl.BlockSpec((1, E), lambda i: (0, 0)),
            pl.BlockSpec((1, E), lambda i: (0, 0)),
            pl.BlockSpec((1, 1), lambda i: (0, 0)),
        ],
        out_shape=[
            jax.ShapeDtypeStruct((T, C), F32),
            jax.ShapeDtypeStruct((T, C), BF),
            jax.ShapeDtypeStruct((T, 1), F32),
            jax.ShapeDtypeStruct((T, 1), F32),
            jax.ShapeDtypeStruct((T, 1), F32),
            jax.ShapeDtypeStruct((T, 1), F32),
            jax.ShapeDtypeStruct((1, E), F32),
            jax.ShapeDtypeStruct((1, E), F32),
            jax.ShapeDtypeStruct((1, 1), F32),
        ],
    )(xf, yh, Wproj.astype(BF), bproj.reshape(1, C),
      ln2_w.reshape(1, C), gate_w, gate_b.reshape(1, E))

    ns_ = T // SH_ROWS
    basev = pl.pallas_call(
        _shared_kernel,
        grid=(ns_,),
        in_specs=[
            pl.BlockSpec((SH_ROWS, C), lambda i: (i, 0)),
            pl.BlockSpec((SH_ROWS, C), lambda i: (i, 0)),
            pl.BlockSpec((C, DFF), lambda i: (0, 0)),
            pl.BlockSpec((1, DFF), lambda i: (0, 0)),
            pl.BlockSpec((DFF, C), lambda i: (0, 0)),
            pl.BlockSpec((1, C), lambda i: (0, 0)),
        ],
        out_specs=pl.BlockSpec((SH_ROWS, C), lambda i: (i, 0)),
        out_shape=jax.ShapeDtypeStruct((T, C), F32),
    )(x1, x2b, Ws1.astype(BF), bs1.reshape(1, DFF),
      Ws2.astype(BF), bs2.reshape(1, C))

    offcnt = jnp.concatenate([offs, counts], axis=1).reshape(2 * E) \
        .astype(jnp.int32)

    hs = pl.pallas_call(
        _moe_kernel,
        grid_spec=pltpu.PrefetchScalarGridSpec(
            num_scalar_prefetch=1,
            grid=(E,),
            in_specs=[
                pl.BlockSpec((T, C), lambda e, s: (0, 0)),
                pl.BlockSpec((T, 1), lambda e, s: (0, 0)),
                pl.BlockSpec((T, 1), lambda e, s: (0, 0)),
                pl.BlockSpec((T, 1), lambda e, s: (0, 0)),
                pl.BlockSpec((T, 1), lambda e, s: (0, 0)),
                pl.BlockSpec((1, C, DFF), lambda e, s: (e, 0, 0)),
                pl.BlockSpec((1, 1, DFF), lambda e, s: (e, 0, 0)),
                pl.BlockSpec((1, DFF, C), lambda e, s: (e, 0, 0)),
                pl.BlockSpec((1, 1, C), lambda e, s: (e, 0, 0)),
            ],
            out_specs=pl.BlockSpec((NS, C), lambda e, s: (0, 0)),
        ),
        out_shape=jax.ShapeDtypeStruct((NS, C), BF),
    )(offcnt, x2b, d1, d2, gw1, gw2, We1, be1.reshape(E, 1, DFF),
      We2, be2.reshape(E, 1, C))

    out = pl.pallas_call(
        _scatter_kernel,
        grid=(NS // SCHUNK,),
        in_specs=[
            pl.BlockSpec((T, 1), lambda c: (0, 0)),
            pl.BlockSpec((T, 1), lambda c: (0, 0)),
            pl.BlockSpec((SCHUNK, C), lambda c: (c, 0)),
            pl.BlockSpec((T, C), lambda c: (0, 0)),
        ],
        out_specs=pl.BlockSpec((T, C), lambda c: (0, 0)),
        out_shape=jax.ShapeDtypeStruct((T, C), F32),
    )(d1, d2, hs, basev)

    return out.reshape(B, T, C), loss.reshape(())
